# Initial kernel scaffold; baseline (speedup 1.0000x reference)
import jax
import jax.numpy as jnp
from jax.experimental import pallas as pl


def kernel(x, edge_index, batch, W1_rel, W1_root, b1, W2_rel, W2_root, b2, W3_rel, W3_root, b3, W_lin, b_lin):
    raise NotImplementedError("write your pallas kernel here")


# speedup vs baseline: 5.9392x; 5.9392x over previous
import functools

import jax
import jax.numpy as jnp
from jax import lax
from jax.experimental import pallas as pl
from jax.experimental.pallas import tpu as pltpu
from jax.experimental.pallas import tpu_sc as plsc

N = 50000
E = 800000
DIN = 100
H = 64
G = 512

HH = H // 2
NS = 16
EPT = E // NS
IB = 125
NI = 4
CH = NI * IB
NCHUNK = EPT // CH
WCH = 1000
NZC = N // WCH

BLK = 2000
NBLK = N // BLK


def _sc_edge_agg(p2, src3, dst3):
    mesh = plsc.VectorSubcoreMesh(core_axis_name="c", subcore_axis_name="s")

    @functools.partial(
        pl.kernel,
        out_type=jax.ShapeDtypeStruct((2, N, HH), jnp.float32),
        mesh=mesh,
        scratch_types=[
            pltpu.VMEM((NI, IB), jnp.int32),
            pltpu.VMEM((NI, IB), jnp.int32),
            pltpu.VMEM((CH, HH), jnp.float32),
            pltpu.VMEM_SHARED((N, HH), jnp.float32),
            pltpu.SemaphoreType.DMA,
        ],
        compiler_params=pltpu.CompilerParams(use_tc_tiling_on_sc=False),
    )
    def k(p_hbm, src_hbm, dst_hbm, out_hbm, sidx, didx, rows, acc, sem):
        c = lax.axis_index("c")
        s = lax.axis_index("s")

        @pl.loop(0, CH)
        def _(r):
            @pl.loop(0, HH, step=16)
            def _(cc):
                rows[r, pl.ds(cc, 16)] = jnp.zeros((16,), jnp.float32)

        @pl.loop(0, (NZC + NS - 1) // NS)
        def _(j):
            k = s + j * NS

            @pl.when(k < NZC)
            def _():
                pltpu.sync_copy(rows, acc.at[pl.ds(k * WCH, CH)])
                pltpu.sync_copy(rows, acc.at[pl.ds(k * WCH + CH, CH)])

        plsc.subcore_barrier()

        @pl.loop(0, NCHUNK)
        def _(i):
            gi = s * NCHUNK + i
            pltpu.sync_copy(src_hbm.at[gi], sidx)
            pltpu.sync_copy(dst_hbm.at[gi], didx)
            for j in range(NI):
                pltpu.async_copy(p_hbm.at[c].at[sidx.at[j]],
                                 rows.at[pl.ds(j * IB, IB)], sem).wait()
            for j in range(NI):
                pltpu.sync_copy(rows.at[pl.ds(j * IB, IB)],
                                acc.at[didx.at[j]], add=True)

        plsc.subcore_barrier()

        @pl.loop(0, (NZC + NS - 1) // NS)
        def _(j):
            k = s + j * NS

            @pl.when(k < NZC)
            def _():
                pltpu.sync_copy(acc.at[pl.ds(k * WCH, WCH)],
                                out_hbm.at[c].at[pl.ds(k * WCH, WCH)])

    return k(p2, src3, dst3)


def _dense_first(x, w_rel, w_root, b):
    def body(x_ref, wr_ref, wo_ref, b_ref, p_ref, r_ref):
        xb = x_ref[...]
        p = jnp.dot(xb, wr_ref[...], preferred_element_type=jnp.float32)
        r = jnp.dot(xb, wo_ref[...], preferred_element_type=jnp.float32)
        p_ref[0, :, :] = p[:, :HH]
        p_ref[1, :, :] = p[:, HH:]
        r_ref[...] = r + b_ref[...]

    d = x.shape[1]
    return pl.pallas_call(
        body,
        grid=(NBLK,),
        in_specs=[
            pl.BlockSpec((BLK, d), lambda i: (i, 0)),
            pl.BlockSpec((d, H), lambda i: (0, 0)),
            pl.BlockSpec((d, H), lambda i: (0, 0)),
            pl.BlockSpec((1, H), lambda i: (0, 0)),
        ],
        out_specs=[
            pl.BlockSpec((2, BLK, HH), lambda i: (0, i, 0)),
            pl.BlockSpec((BLK, H), lambda i: (i, 0)),
        ],
        out_shape=[
            jax.ShapeDtypeStruct((2, N, HH), jnp.float32),
            jax.ShapeDtypeStruct((N, H), jnp.float32),
        ],
    )(x, w_rel, w_root, b)


def _dense_mid(agg, r, w_rel, w_root, b):
    def body(agg_ref, r_ref, wr_ref, wo_ref, b_ref, p_ref, rout_ref):
        h = jnp.concatenate([agg_ref[0, :, :], agg_ref[1, :, :]], axis=1)
        h = jnp.maximum(h + r_ref[...], 0.0)
        p = jnp.dot(h, wr_ref[...], preferred_element_type=jnp.float32)
        rn = jnp.dot(h, wo_ref[...], preferred_element_type=jnp.float32)
        p_ref[0, :, :] = p[:, :HH]
        p_ref[1, :, :] = p[:, HH:]
        rout_ref[...] = rn + b_ref[...]

    return pl.pallas_call(
        body,
        grid=(NBLK,),
        in_specs=[
            pl.BlockSpec((2, BLK, HH), lambda i: (0, i, 0)),
            pl.BlockSpec((BLK, H), lambda i: (i, 0)),
            pl.BlockSpec((H, H), lambda i: (0, 0)),
            pl.BlockSpec((H, H), lambda i: (0, 0)),
            pl.BlockSpec((1, H), lambda i: (0, 0)),
        ],
        out_specs=[
            pl.BlockSpec((2, BLK, HH), lambda i: (0, i, 0)),
            pl.BlockSpec((BLK, H), lambda i: (i, 0)),
        ],
        out_shape=[
            jax.ShapeDtypeStruct((2, N, HH), jnp.float32),
            jax.ShapeDtypeStruct((N, H), jnp.float32),
        ],
    )(agg, r, w_rel, w_root, b)


def _pool_head(agg, r, batch3, w_lin, b_lin):
    def body(agg_ref, r_ref, batch_ref, wl_ref, bl_ref, out_ref, sums, counts):
        i = pl.program_id(0)

        @pl.when(i == 0)
        def _():
            sums[...] = jnp.zeros_like(sums)
            counts[...] = jnp.zeros_like(counts)

        h = jnp.concatenate([agg_ref[0, :, :], agg_ref[1, :, :]], axis=1)
        h = h + r_ref[...]
        bids = batch_ref[0, 0, :]
        onehot = (bids[None, :] ==
                  lax.broadcasted_iota(jnp.int32, (G, 1), 0)).astype(jnp.float32)
        sums[...] += jnp.dot(onehot, h, preferred_element_type=jnp.float32)
        counts[...] += jnp.sum(onehot, axis=1, keepdims=True)

        @pl.when(i == pl.num_programs(0) - 1)
        def _():
            pooled = sums[...] / jnp.maximum(counts[...], 1.0)
            out_ref[...] = (jnp.dot(pooled, wl_ref[...],
                                    preferred_element_type=jnp.float32)
                            + bl_ref[...])

    return pl.pallas_call(
        body,
        grid=(NBLK,),
        in_specs=[
            pl.BlockSpec((2, BLK, HH), lambda i: (0, i, 0)),
            pl.BlockSpec((BLK, H), lambda i: (i, 0)),
            pl.BlockSpec((1, 1, BLK), lambda i: (i, 0, 0)),
            pl.BlockSpec((H, 2), lambda i: (0, 0)),
            pl.BlockSpec((1, 2), lambda i: (0, 0)),
        ],
        out_specs=pl.BlockSpec((G, 2), lambda i: (0, 0)),
        out_shape=jax.ShapeDtypeStruct((G, 2), jnp.float32),
        scratch_shapes=[
            pltpu.VMEM((G, H), jnp.float32),
            pltpu.VMEM((G, 1), jnp.float32),
        ],
    )(agg, r, batch3, w_lin, b_lin)


def kernel(x, edge_index, batch, W1_rel, W1_root, b1, W2_rel, W2_root, b2,
           W3_rel, W3_root, b3, W_lin, b_lin):
    src3 = edge_index[0].reshape(E // CH, NI, IB)
    dst3 = edge_index[1].reshape(E // CH, NI, IB)
    batch3 = batch.reshape(NBLK, 1, BLK)

    p1, r1 = _dense_first(x, W1_rel, W1_root, b1.reshape(1, H))
    agg1 = _sc_edge_agg(p1, src3, dst3)
    p2, r2 = _dense_mid(agg1, r1, W2_rel, W2_root, b2.reshape(1, H))
    agg2 = _sc_edge_agg(p2, src3, dst3)
    p3, r3 = _dense_mid(agg2, r2, W3_rel, W3_root, b3.reshape(1, H))
    agg3 = _sc_edge_agg(p3, src3, dst3)
    return _pool_head(agg3, r3, batch3, W_lin, b_lin.reshape(1, 2))


# speedup vs baseline: 10.4504x; 1.7596x over previous
import functools

import jax
import jax.numpy as jnp
from jax import lax
from jax.experimental import pallas as pl
from jax.experimental.pallas import tpu as pltpu
from jax.experimental.pallas import tpu_sc as plsc

N = 50000
E = 800000
DIN = 100
H = 64
G = 512

HH = H // 2
NS = 16
EPT = E // NS
IB = 125
NI = 2
CH = NI * IB
NCHUNK = EPT // CH
WCH = 1000
NZC = N // WCH

BLK = 2000
NBLK = N // BLK


def _sc_edge_agg(p2, src3, dst3):
    mesh = plsc.VectorSubcoreMesh(core_axis_name="c", subcore_axis_name="s")

    @functools.partial(
        pl.kernel,
        out_type=jax.ShapeDtypeStruct((2, N, HH), jnp.float32),
        mesh=mesh,
        scratch_types=[
            pltpu.VMEM((NI, IB), jnp.int32),
            pltpu.VMEM((NI, IB), jnp.int32),
            pltpu.VMEM((NI, IB), jnp.int32),
            pltpu.VMEM((NI, IB), jnp.int32),
            pltpu.VMEM((CH, HH), jnp.float32),
            pltpu.VMEM((CH, HH), jnp.float32),
            pltpu.VMEM_SHARED((N, HH), jnp.float32),
            pltpu.SemaphoreType.DMA,
            pltpu.SemaphoreType.DMA,
            pltpu.SemaphoreType.DMA,
            pltpu.SemaphoreType.DMA,
        ],
        compiler_params=pltpu.CompilerParams(use_tc_tiling_on_sc=False),
    )
    def k(p_hbm, src_hbm, dst_hbm, out_hbm, sidx0, didx0, sidx1, didx1,
          rows0, rows1, acc, semI0, semI1, semG0, semG1):
        c = lax.axis_index("c")
        s = lax.axis_index("s")

        @pl.loop(0, CH)
        def _(r):
            @pl.loop(0, HH, step=16)
            def _(cc):
                rows0[r, pl.ds(cc, 16)] = jnp.zeros((16,), jnp.float32)

        @pl.loop(0, (NZC + NS - 1) // NS)
        def _(j):
            kk = s + j * NS

            @pl.when(kk < NZC)
            def _():
                for q in range(WCH // CH):
                    pltpu.sync_copy(rows0, acc.at[pl.ds(kk * WCH + q * CH, CH)])

        plsc.subcore_barrier()

        e0 = s * NCHUNK

        def fire_idx(i, sb, db, sem):
            pltpu.async_copy(src_hbm.at[e0 + i], sb, sem)
            pltpu.async_copy(dst_hbm.at[e0 + i], db, sem)

        def wait_idx(sb, db, sem):
            pltpu.make_async_copy(src_hbm.at[0], sb, sem).wait()
            pltpu.make_async_copy(dst_hbm.at[0], db, sem).wait()

        def fire_gather(sb, rows_b, sem):
            for j in range(NI):
                pltpu.async_copy(p_hbm.at[c].at[sb.at[j]],
                                 rows_b.at[pl.ds(j * IB, IB)], sem)

        def wait_gather(sb, rows_b, sem):
            for j in range(NI):
                pltpu.make_async_copy(p_hbm.at[c].at[sb.at[j]],
                                      rows_b.at[pl.ds(j * IB, IB)], sem).wait()

        def scatter(db, rows_b):
            for j in range(NI):
                pltpu.sync_copy(rows_b.at[pl.ds(j * IB, IB)],
                                acc.at[db.at[j]], add=True)

        fire_idx(0, sidx0, didx0, semI0)
        fire_idx(1, sidx1, didx1, semI1)
        wait_idx(sidx0, didx0, semI0)
        fire_gather(sidx0, rows0, semG0)

        @pl.loop(0, NCHUNK // 2)
        def _(t):
            a = 2 * t
            wait_idx(sidx1, didx1, semI1)
            fire_gather(sidx1, rows1, semG1)
            wait_gather(sidx0, rows0, semG0)
            scatter(didx0, rows0)

            @pl.when(a + 2 < NCHUNK)
            def _():
                fire_idx(a + 2, sidx0, didx0, semI0)

            wait_gather(sidx1, rows1, semG1)

            @pl.when(a + 2 < NCHUNK)
            def _():
                wait_idx(sidx0, didx0, semI0)
                fire_gather(sidx0, rows0, semG0)

            scatter(didx1, rows1)

            @pl.when(a + 3 < NCHUNK)
            def _():
                fire_idx(a + 3, sidx1, didx1, semI1)

        plsc.subcore_barrier()

        @pl.loop(0, (NZC + NS - 1) // NS)
        def _(j):
            k = s + j * NS

            @pl.when(k < NZC)
            def _():
                pltpu.sync_copy(acc.at[pl.ds(k * WCH, WCH)],
                                out_hbm.at[c].at[pl.ds(k * WCH, WCH)])

    return k(p2, src3, dst3)


def _dense_first(x, w_rel, w_root, b):
    def body(x_ref, wr_ref, wo_ref, b_ref, p_ref, r_ref):
        xb = x_ref[...]
        p = jnp.dot(xb, wr_ref[...], preferred_element_type=jnp.float32)
        r = jnp.dot(xb, wo_ref[...], preferred_element_type=jnp.float32)
        p_ref[0, :, :] = p[:, :HH]
        p_ref[1, :, :] = p[:, HH:]
        r_ref[...] = r + b_ref[...]

    d = x.shape[1]
    return pl.pallas_call(
        body,
        grid=(NBLK,),
        in_specs=[
            pl.BlockSpec((BLK, d), lambda i: (i, 0)),
            pl.BlockSpec((d, H), lambda i: (0, 0)),
            pl.BlockSpec((d, H), lambda i: (0, 0)),
            pl.BlockSpec((1, H), lambda i: (0, 0)),
        ],
        out_specs=[
            pl.BlockSpec((2, BLK, HH), lambda i: (0, i, 0)),
            pl.BlockSpec((BLK, H), lambda i: (i, 0)),
        ],
        out_shape=[
            jax.ShapeDtypeStruct((2, N, HH), jnp.float32),
            jax.ShapeDtypeStruct((N, H), jnp.float32),
        ],
    )(x, w_rel, w_root, b)


def _dense_mid(agg, r, w_rel, w_root, b):
    def body(agg_ref, r_ref, wr_ref, wo_ref, b_ref, p_ref, rout_ref):
        h = jnp.concatenate([agg_ref[0, :, :], agg_ref[1, :, :]], axis=1)
        h = jnp.maximum(h + r_ref[...], 0.0)
        p = jnp.dot(h, wr_ref[...], preferred_element_type=jnp.float32)
        rn = jnp.dot(h, wo_ref[...], preferred_element_type=jnp.float32)
        p_ref[0, :, :] = p[:, :HH]
        p_ref[1, :, :] = p[:, HH:]
        rout_ref[...] = rn + b_ref[...]

    return pl.pallas_call(
        body,
        grid=(NBLK,),
        in_specs=[
            pl.BlockSpec((2, BLK, HH), lambda i: (0, i, 0)),
            pl.BlockSpec((BLK, H), lambda i: (i, 0)),
            pl.BlockSpec((H, H), lambda i: (0, 0)),
            pl.BlockSpec((H, H), lambda i: (0, 0)),
            pl.BlockSpec((1, H), lambda i: (0, 0)),
        ],
        out_specs=[
            pl.BlockSpec((2, BLK, HH), lambda i: (0, i, 0)),
            pl.BlockSpec((BLK, H), lambda i: (i, 0)),
        ],
        out_shape=[
            jax.ShapeDtypeStruct((2, N, HH), jnp.float32),
            jax.ShapeDtypeStruct((N, H), jnp.float32),
        ],
    )(agg, r, w_rel, w_root, b)


def _pool_head(agg, r, batch3, w_lin, b_lin):
    def body(agg_ref, r_ref, batch_ref, wl_ref, bl_ref, out_ref, sums, counts):
        i = pl.program_id(0)

        @pl.when(i == 0)
        def _():
            sums[...] = jnp.zeros_like(sums)
            counts[...] = jnp.zeros_like(counts)

        h = jnp.concatenate([agg_ref[0, :, :], agg_ref[1, :, :]], axis=1)
        h = h + r_ref[...]
        bids = batch_ref[0, 0, :]
        onehot = (bids[None, :] ==
                  lax.broadcasted_iota(jnp.int32, (G, 1), 0)).astype(jnp.float32)
        sums[...] += jnp.dot(onehot, h, preferred_element_type=jnp.float32)
        counts[...] += jnp.sum(onehot, axis=1, keepdims=True)

        @pl.when(i == pl.num_programs(0) - 1)
        def _():
            pooled = sums[...] / jnp.maximum(counts[...], 1.0)
            out_ref[...] = (jnp.dot(pooled, wl_ref[...],
                                    preferred_element_type=jnp.float32)
                            + bl_ref[...])

    return pl.pallas_call(
        body,
        grid=(NBLK,),
        in_specs=[
            pl.BlockSpec((2, BLK, HH), lambda i: (0, i, 0)),
            pl.BlockSpec((BLK, H), lambda i: (i, 0)),
            pl.BlockSpec((1, 1, BLK), lambda i: (i, 0, 0)),
            pl.BlockSpec((H, 2), lambda i: (0, 0)),
            pl.BlockSpec((1, 2), lambda i: (0, 0)),
        ],
        out_specs=pl.BlockSpec((G, 2), lambda i: (0, 0)),
        out_shape=jax.ShapeDtypeStruct((G, 2), jnp.float32),
        scratch_shapes=[
            pltpu.VMEM((G, H), jnp.float32),
            pltpu.VMEM((G, 1), jnp.float32),
        ],
    )(agg, r, batch3, w_lin, b_lin)


def kernel(x, edge_index, batch, W1_rel, W1_root, b1, W2_rel, W2_root, b2,
           W3_rel, W3_root, b3, W_lin, b_lin):
    src3 = edge_index[0].reshape(E // CH, NI, IB)
    dst3 = edge_index[1].reshape(E // CH, NI, IB)
    batch3 = batch.reshape(NBLK, 1, BLK)

    p1, r1 = _dense_first(x, W1_rel, W1_root, b1.reshape(1, H))
    agg1 = _sc_edge_agg(p1, src3, dst3)
    p2, r2 = _dense_mid(agg1, r1, W2_rel, W2_root, b2.reshape(1, H))
    agg2 = _sc_edge_agg(p2, src3, dst3)
    p3, r3 = _dense_mid(agg2, r2, W3_rel, W3_root, b3.reshape(1, H))
    agg3 = _sc_edge_agg(p3, src3, dst3)
    return _pool_head(agg3, r3, batch3, W_lin, b_lin.reshape(1, 2))
